# baseline (speedup 1.0000x reference)
import functools

import jax
import jax.numpy as jnp
from jax import lax
from jax.experimental import pallas as pl
from jax.experimental.pallas import tpu as pltpu
from jax.experimental.pallas import tpu_sc as plsc

NUM = 8192
HALF = NUM // 2
DIM = 32
TILE_M = 512


def _vq_idx_kernel(x_ref, e_ref, idx_ref):
    x = x_ref[...]
    e = e_ref[...]
    xsq = jnp.sum(x * x, axis=1, keepdims=True)
    esq = jnp.sum(e * e, axis=1)
    mm2 = jax.lax.dot_general(
        (x + x).astype(jnp.bfloat16), e.astype(jnp.bfloat16),
        (((1,), (1,)), ((), ())),
        preferred_element_type=jnp.float32)
    d = (xsq + esq[None, :]) - mm2

    d_lo, d_hi = d[:, :HALF], d[:, HALF:]
    m_lo = jnp.min(d_lo, axis=1, keepdims=True)
    m_hi = jnp.min(d_hi, axis=1, keepdims=True)
    m_lo_q = m_lo.astype(jnp.bfloat16).astype(jnp.float32)
    take_hi = m_hi < m_lo_q
    d_sel = jnp.where(take_hi, d_hi, d_lo)
    m_sel = jnp.where(take_hi, m_hi, m_lo)
    iota = jax.lax.broadcasted_iota(jnp.int32, d_sel.shape, 1)
    idx_h = jnp.min(jnp.where(d_sel == m_sel, iota, HALF), axis=1)
    idx_ref[0, 0, :] = idx_h + jnp.where(take_hi[:, 0], HALF, 0)


_info = plsc.get_sparse_core_info()
_NC, _NS = _info.num_cores, _info.num_subcores
_NW = _NC * _NS


_LANES = 128
_B = 65536
_B_PER_W = _B // _NW
_CHUNK = 512


@functools.partial(
    pl.kernel,
    mesh=plsc.VectorSubcoreMesh(core_axis_name="c", subcore_axis_name="s"),
    out_type=jax.ShapeDtypeStruct((_B, _LANES), jnp.float32),
    scratch_types=[
        pltpu.VMEM((_CHUNK,), jnp.int32),
        pltpu.VMEM((_CHUNK, _LANES), jnp.float32),
        pltpu.SemaphoreType.DMA,
    ],
)
def _sc_gather(table_hbm, idx_hbm, out_hbm, idx_v, rows_v, sem):
    wid = lax.axis_index("s") * _NC + lax.axis_index("c")
    base = wid * _B_PER_W
    for c in range(_B_PER_W // _CHUNK):
        off = base + c * _CHUNK
        pltpu.sync_copy(idx_hbm.at[pl.ds(off, _CHUNK)], idx_v)
        pltpu.async_copy(table_hbm.at[idx_v], rows_v, sem).wait()
        pltpu.sync_copy(rows_v, out_hbm.at[pl.ds(off, _CHUNK)])


def kernel(inputs, embeddings):
    input_shape = inputs.shape
    flat = inputs.reshape(-1, DIM)
    n = flat.shape[0]
    nblk = n // TILE_M
    idx = pl.pallas_call(
        _vq_idx_kernel,
        grid=(nblk,),
        in_specs=[
            pl.BlockSpec((TILE_M, DIM), lambda i: (i, 0)),
            pl.BlockSpec((NUM, DIM), lambda i: (0, 0)),
        ],
        out_specs=pl.BlockSpec((1, 1, TILE_M), lambda i: (i, 0, 0)),
        out_shape=jax.ShapeDtypeStruct((nblk, 1, TILE_M), jnp.int32),
    )(flat, embeddings)
    table = jnp.pad(embeddings, ((0, 0), (0, _LANES - DIM)))
    q = _sc_gather(table, idx.reshape(n))
    return q[:, :DIM].reshape(input_shape)


# speedup vs baseline: 1.0688x; 1.0688x over previous
import functools

import jax
import jax.numpy as jnp
from jax import lax
from jax.experimental import pallas as pl
from jax.experimental.pallas import tpu as pltpu
from jax.experimental.pallas import tpu_sc as plsc

NUM = 8192
HALF = NUM // 2
DIM = 32
TILE_M = 512


def _vq_idx_kernel(x_ref, e_ref, idx_ref):
    x = x_ref[...]
    e = e_ref[...]
    xsq = jnp.sum(x * x, axis=1, keepdims=True)
    esq = jnp.sum(e * e, axis=1)
    mm = jax.lax.dot_general(
        x.astype(jnp.bfloat16), e.astype(jnp.bfloat16),
        (((1,), (1,)), ((), ())),
        preferred_element_type=jnp.float32)
    d = (xsq + esq[None, :]) - 2.0 * mm

    d_lo, d_hi = d[:, :HALF], d[:, HALF:]
    m_lo = jnp.min(d_lo, axis=1, keepdims=True)
    m_hi = jnp.min(d_hi, axis=1, keepdims=True)
    idx_lo = jnp.argmin(d_lo, axis=1).astype(jnp.int32)
    idx_hi = jnp.argmin(d_hi, axis=1).astype(jnp.int32) + HALF
    m_lo_q = m_lo.astype(jnp.bfloat16).astype(jnp.float32)
    take_hi = m_hi[:, 0] < m_lo_q[:, 0]
    idx_ref[0, 0, :] = jnp.where(take_hi, idx_hi, idx_lo)


_info = plsc.get_sparse_core_info()
_NC, _NS = _info.num_cores, _info.num_subcores
_NW = _NC * _NS


_LANES = 128
_B = 65536
_B_PER_W = _B // _NW
_CHUNK = 512


@functools.partial(
    pl.kernel,
    mesh=plsc.VectorSubcoreMesh(core_axis_name="c", subcore_axis_name="s"),
    out_type=jax.ShapeDtypeStruct((_B, _LANES), jnp.float32),
    scratch_types=[
        pltpu.VMEM((_CHUNK,), jnp.int32),
        pltpu.VMEM((_CHUNK, _LANES), jnp.float32),
        pltpu.SemaphoreType.DMA,
    ],
)
def _sc_gather(table_hbm, idx_hbm, out_hbm, idx_v, rows_v, sem):
    wid = lax.axis_index("s") * _NC + lax.axis_index("c")
    base = wid * _B_PER_W
    for c in range(_B_PER_W // _CHUNK):
        off = base + c * _CHUNK
        pltpu.sync_copy(idx_hbm.at[pl.ds(off, _CHUNK)], idx_v)
        pltpu.async_copy(table_hbm.at[idx_v], rows_v, sem).wait()
        pltpu.sync_copy(rows_v, out_hbm.at[pl.ds(off, _CHUNK)])


def kernel(inputs, embeddings):
    input_shape = inputs.shape
    flat = inputs.reshape(-1, DIM)
    n = flat.shape[0]
    nblk = n // TILE_M
    idx = pl.pallas_call(
        _vq_idx_kernel,
        grid=(nblk,),
        in_specs=[
            pl.BlockSpec((TILE_M, DIM), lambda i: (i, 0)),
            pl.BlockSpec((NUM, DIM), lambda i: (0, 0)),
        ],
        out_specs=pl.BlockSpec((1, 1, TILE_M), lambda i: (i, 0, 0)),
        out_shape=jax.ShapeDtypeStruct((nblk, 1, TILE_M), jnp.int32),
    )(flat, embeddings)
    table = jnp.pad(embeddings, ((0, 0), (0, _LANES - DIM)))
    q = _sc_gather(table, idx.reshape(n))
    return q[:, :DIM].reshape(input_shape)
